# baseline (speedup 1.0000x reference)
import functools

import jax
import jax.numpy as jnp
from jax import lax
from jax.experimental import pallas as pl
from jax.experimental.pallas import tpu as pltpu
from jax.experimental.pallas import tpu_sc as plsc

B = 16384
TYPE_VOCAB = 1000
DOMAIN_DIM = 16
TYPE_DIM = 8
DAY_DIM = 4
HIDDEN = 128

NC = 2
NS = 16
LANES = 16
NW = NC * NS
BPW = B // NW
GROUPS = BPW // LANES


def _sc_gather_body(dom_tab8, typ_tab, day_tab, dom_id, typ_id, day_id,
                    dm_out, t_out, d_out,
                    idx_v, idx8_v, rows8_v, tid_v, did_v, ttab_v, dtab_v,
                    dmblk_v, tblk_v, dblk_v, sem):
    wid = lax.axis_index("s") * NC + lax.axis_index("c")
    base = wid * BPW

    pltpu.sync_copy(dom_id.at[pl.ds(base, BPW)], idx_v)
    pltpu.sync_copy(typ_id.at[pl.ds(base, BPW)], tid_v)
    pltpu.sync_copy(day_id.at[pl.ds(base, BPW)], did_v)
    pltpu.sync_copy(typ_tab, ttab_v)
    pltpu.sync_copy(day_tab, dtab_v)

    for g in range(GROUPS):
        ids = idx_v[pl.ds(g * LANES, LANES)]
        idx8_v[pl.ds(g * LANES, LANES)] = lax.shift_right_logical(ids, 3)
    pltpu.async_copy(dom_tab8.at[idx8_v], rows8_v, sem).wait()

    lane = lax.iota(jnp.int32, LANES)
    for g in range(GROUPS):
        ids = idx_v[pl.ds(g * LANES, LANES)]
        row = g * LANES + lane
        col = (ids & 7) * DOMAIN_DIM
        dst = row * DOMAIN_DIM
        for j in range(DOMAIN_DIM):
            v = plsc.load_gather(rows8_v, [row, col + j])
            plsc.store_scatter(dmblk_v, [dst + j], v)

        tids = tid_v[pl.ds(g * LANES, LANES)]
        tdst = (g * LANES + lane) * TYPE_DIM
        tsrc = tids * TYPE_DIM
        for j in range(TYPE_DIM):
            v = plsc.load_gather(ttab_v, [tsrc + j])
            plsc.store_scatter(tblk_v, [tdst + j], v)

        dids = did_v[pl.ds(g * LANES, LANES)]
        ddst = (g * LANES + lane) * DAY_DIM
        dsrc = dids * DAY_DIM
        for j in range(DAY_DIM):
            v = plsc.load_gather(dtab_v, [dsrc + j])
            plsc.store_scatter(dblk_v, [ddst + j], v)

    pltpu.sync_copy(dmblk_v,
                    dm_out.at[pl.ds(base * DOMAIN_DIM, BPW * DOMAIN_DIM)])
    pltpu.sync_copy(tblk_v, t_out.at[pl.ds(base * TYPE_DIM, BPW * TYPE_DIM)])
    pltpu.sync_copy(dblk_v, d_out.at[pl.ds(base * DAY_DIM, BPW * DAY_DIM)])


_sc_gather = functools.partial(
    pl.kernel,
    out_type=(
        jax.ShapeDtypeStruct((B * DOMAIN_DIM,), jnp.float32),
        jax.ShapeDtypeStruct((B * TYPE_DIM,), jnp.float32),
        jax.ShapeDtypeStruct((B * DAY_DIM,), jnp.float32),
    ),
    mesh=plsc.VectorSubcoreMesh(core_axis_name="c", subcore_axis_name="s"),
    compiler_params=pltpu.CompilerParams(needs_layout_passes=False,
                                         use_tc_tiling_on_sc=False),
    scratch_types=(
        pltpu.VMEM((BPW,), jnp.int32),
        pltpu.VMEM((BPW,), jnp.int32),
        pltpu.VMEM((BPW, 128), jnp.float32),
        pltpu.VMEM((BPW,), jnp.int32),
        pltpu.VMEM((BPW,), jnp.int32),
        pltpu.VMEM((TYPE_VOCAB * TYPE_DIM,), jnp.float32),
        pltpu.VMEM((8 * DAY_DIM,), jnp.float32),
        pltpu.VMEM((BPW * DOMAIN_DIM,), jnp.float32),
        pltpu.VMEM((BPW * TYPE_DIM,), jnp.float32),
        pltpu.VMEM((BPW * DAY_DIM,), jnp.float32),
        pltpu.SemaphoreType.DMA,
    ),
)(_sc_gather_body)


BLK = 2048


def _mlp_body(dm_ref, t_ref, d_ref, nf_ref,
              w1t_ref, w1d_ref, w1dm_ref, w1nf_ref,
              b1_ref, w2_ref, b2_ref, out_ref):
    h = jnp.dot(t_ref[...], w1t_ref[...], preferred_element_type=jnp.float32)
    h += jnp.dot(d_ref[...], w1d_ref[...], preferred_element_type=jnp.float32)
    h += jnp.dot(dm_ref[...], w1dm_ref[...], preferred_element_type=jnp.float32)
    h += jnp.dot(nf_ref[...], w1nf_ref[...], preferred_element_type=jnp.float32)
    h = jnp.maximum(h + b1_ref[...], 0.0)
    out = jnp.dot(h, w2_ref[...], preferred_element_type=jnp.float32)
    out_ref[...] = out + b2_ref[...]


def kernel(type_id, day_of_week_id, domain_id, hour_of_day, karma, descendants,
           type_table, day_table, domain_table, W1, b1, W2, b2):
    type_id = type_id.astype(jnp.int32)
    day_of_week_id = day_of_week_id.astype(jnp.int32)
    domain_id = domain_id.astype(jnp.int32)
    day_pad = jnp.pad(day_table, ((0, 8 - day_table.shape[0]), (0, 0)))

    dm, t, d = _sc_gather(domain_table.reshape(-1, 128),
                          type_table.reshape(-1), day_pad.reshape(-1),
                          domain_id, type_id, day_of_week_id)
    dm = dm.reshape(B, DOMAIN_DIM)
    t = t.reshape(B, TYPE_DIM)
    d = d.reshape(B, DAY_DIM)

    nf = jnp.stack([hour_of_day, karma, descendants,
                    jnp.zeros((B,), jnp.float32)], axis=1)
    w1nf = jnp.concatenate(
        [W1[28:31], jnp.zeros((1, HIDDEN), jnp.float32)], axis=0)

    row_blk = lambda i: (i, 0)
    whole = lambda i: (0, 0)
    out = pl.pallas_call(
        _mlp_body,
        grid=(B // BLK,),
        in_specs=[
            pl.BlockSpec((BLK, DOMAIN_DIM), row_blk),
            pl.BlockSpec((BLK, TYPE_DIM), row_blk),
            pl.BlockSpec((BLK, DAY_DIM), row_blk),
            pl.BlockSpec((BLK, 4), row_blk),
            pl.BlockSpec((TYPE_DIM, HIDDEN), whole),
            pl.BlockSpec((DAY_DIM, HIDDEN), whole),
            pl.BlockSpec((DOMAIN_DIM, HIDDEN), whole),
            pl.BlockSpec((4, HIDDEN), whole),
            pl.BlockSpec((1, HIDDEN), whole),
            pl.BlockSpec((HIDDEN, 1), whole),
            pl.BlockSpec((1, 1), whole),
        ],
        out_specs=pl.BlockSpec((BLK, 1), row_blk),
        out_shape=jax.ShapeDtypeStruct((B, 1), jnp.float32),
    )(
        dm, t, d, nf,
        W1[0:TYPE_DIM],
        W1[TYPE_DIM:TYPE_DIM + DAY_DIM],
        W1[TYPE_DIM + DAY_DIM:TYPE_DIM + DAY_DIM + DOMAIN_DIM],
        w1nf,
        b1[None, :], W2, b2[None, :],
    )
    return out[:, 0]


# speedup vs baseline: 3.4993x; 3.4993x over previous
import functools

import jax
import jax.numpy as jnp
from jax import lax
from jax.experimental import pallas as pl
from jax.experimental.pallas import tpu as pltpu
from jax.experimental.pallas import tpu_sc as plsc

B = 16384
TYPE_VOCAB = 1000
DOMAIN_DIM = 16
TYPE_DIM = 8
DAY_DIM = 4
HIDDEN = 128

NC = 2
NS = 16
LANES = 16
NW = NC * NS
BPW = B // NW
GROUPS = BPW // LANES
DMA_CHUNK = 16


def _sc_gather_body(dom_t, typ_tab, day_tab, dom_id, typ_id, day_id,
                    dm_out, t_out, d_out,
                    didx_v, tiles_v, tid_v, did_v, ttab_v, dtab_v,
                    dmblk_v, tblk_v, dblk_v, sem):
    wid = lax.axis_index("s") * NC + lax.axis_index("c")
    base = wid * BPW

    pltpu.sync_copy(dom_id.at[pl.ds(base, BPW)], didx_v)
    pltpu.sync_copy(typ_id.at[pl.ds(base, BPW)], tid_v)
    pltpu.sync_copy(day_id.at[pl.ds(base, BPW)], did_v)
    pltpu.sync_copy(typ_tab, ttab_v)
    pltpu.sync_copy(day_tab, dtab_v)

    lane = lax.iota(jnp.int32, LANES)

    def dma_chunk(ch, _):
        ids = didx_v[pl.ds(ch * DMA_CHUNK, DMA_CHUNK)]
        ks = lax.shift_right_logical(ids, 7)
        ls = ids & 127
        copies = []
        for j in range(DMA_CHUNK):
            off = pl.multiple_of(ks[j] * 128, 128)
            copies.append(pltpu.async_copy(
                dom_t.at[:, pl.ds(off, 128)], tiles_v.at[j], sem))
        for c in copies:
            c.wait()
        for j in range(DMA_CHUNK):
            i = ch * DMA_CHUNK + j
            v = plsc.load_gather(
                tiles_v, [jnp.full((LANES,), j, jnp.int32), lane,
                          jnp.broadcast_to(ls[j], (LANES,))])
            dmblk_v[pl.ds(i * DOMAIN_DIM, DOMAIN_DIM)] = v
        return ()

    lax.fori_loop(0, BPW // DMA_CHUNK, dma_chunk, (), unroll=False)

    for g in range(GROUPS):
        tids = tid_v[pl.ds(g * LANES, LANES)]
        tdst = (g * LANES + lane) * TYPE_DIM
        tsrc = tids * TYPE_DIM
        for j in range(TYPE_DIM):
            v = plsc.load_gather(ttab_v, [tsrc + j])
            plsc.store_scatter(tblk_v, [tdst + j], v)

        dids = did_v[pl.ds(g * LANES, LANES)]
        ddst = (g * LANES + lane) * DAY_DIM
        dsrc = dids * DAY_DIM
        for j in range(DAY_DIM):
            v = plsc.load_gather(dtab_v, [dsrc + j])
            plsc.store_scatter(dblk_v, [ddst + j], v)

    pltpu.sync_copy(dmblk_v,
                    dm_out.at[pl.ds(base * DOMAIN_DIM, BPW * DOMAIN_DIM)])
    pltpu.sync_copy(tblk_v, t_out.at[pl.ds(base * TYPE_DIM, BPW * TYPE_DIM)])
    pltpu.sync_copy(dblk_v, d_out.at[pl.ds(base * DAY_DIM, BPW * DAY_DIM)])


_sc_gather = functools.partial(
    pl.kernel,
    out_type=(
        jax.ShapeDtypeStruct((B * DOMAIN_DIM,), jnp.float32),
        jax.ShapeDtypeStruct((B * TYPE_DIM,), jnp.float32),
        jax.ShapeDtypeStruct((B * DAY_DIM,), jnp.float32),
    ),
    mesh=plsc.VectorSubcoreMesh(core_axis_name="c", subcore_axis_name="s"),
    compiler_params=pltpu.CompilerParams(needs_layout_passes=False,
                                         use_tc_tiling_on_sc=True),
    scratch_types=(
        pltpu.VMEM((BPW,), jnp.int32),
        pltpu.VMEM((DMA_CHUNK, DOMAIN_DIM, 128), jnp.float32),
        pltpu.VMEM((BPW,), jnp.int32),
        pltpu.VMEM((BPW,), jnp.int32),
        pltpu.VMEM((TYPE_VOCAB * TYPE_DIM,), jnp.float32),
        pltpu.VMEM((8 * DAY_DIM,), jnp.float32),
        pltpu.VMEM((BPW * DOMAIN_DIM,), jnp.float32),
        pltpu.VMEM((BPW * TYPE_DIM,), jnp.float32),
        pltpu.VMEM((BPW * DAY_DIM,), jnp.float32),
        pltpu.SemaphoreType.DMA,
    ),
)(_sc_gather_body)


BLK = 2048


def _mlp_body(dm_ref, t_ref, d_ref, nf_ref,
              w1t_ref, w1d_ref, w1dm_ref, w1nf_ref,
              b1_ref, w2_ref, b2_ref, out_ref):
    h = jnp.dot(t_ref[...], w1t_ref[...], preferred_element_type=jnp.float32)
    h += jnp.dot(d_ref[...], w1d_ref[...], preferred_element_type=jnp.float32)
    h += jnp.dot(dm_ref[...], w1dm_ref[...], preferred_element_type=jnp.float32)
    h += jnp.dot(nf_ref[...], w1nf_ref[...], preferred_element_type=jnp.float32)
    h = jnp.maximum(h + b1_ref[...], 0.0)
    out = jnp.dot(h, w2_ref[...], preferred_element_type=jnp.float32)
    out_ref[...] = out + b2_ref[...]


def kernel(type_id, day_of_week_id, domain_id, hour_of_day, karma, descendants,
           type_table, day_table, domain_table, W1, b1, W2, b2):
    type_id = type_id.astype(jnp.int32)
    day_of_week_id = day_of_week_id.astype(jnp.int32)
    domain_id = domain_id.astype(jnp.int32)
    day_pad = jnp.pad(day_table, ((0, 8 - day_table.shape[0]), (0, 0)))

    dm, t, d = _sc_gather(domain_table.T,
                          type_table.reshape(-1), day_pad.reshape(-1),
                          domain_id, type_id, day_of_week_id)
    dm = dm.reshape(B, DOMAIN_DIM)
    t = t.reshape(B, TYPE_DIM)
    d = d.reshape(B, DAY_DIM)

    nf = jnp.stack([hour_of_day, karma, descendants,
                    jnp.zeros((B,), jnp.float32)], axis=1)
    w1nf = jnp.concatenate(
        [W1[28:31], jnp.zeros((1, HIDDEN), jnp.float32)], axis=0)

    row_blk = lambda i: (i, 0)
    whole = lambda i: (0, 0)
    out = pl.pallas_call(
        _mlp_body,
        grid=(B // BLK,),
        in_specs=[
            pl.BlockSpec((BLK, DOMAIN_DIM), row_blk),
            pl.BlockSpec((BLK, TYPE_DIM), row_blk),
            pl.BlockSpec((BLK, DAY_DIM), row_blk),
            pl.BlockSpec((BLK, 4), row_blk),
            pl.BlockSpec((TYPE_DIM, HIDDEN), whole),
            pl.BlockSpec((DAY_DIM, HIDDEN), whole),
            pl.BlockSpec((DOMAIN_DIM, HIDDEN), whole),
            pl.BlockSpec((4, HIDDEN), whole),
            pl.BlockSpec((1, HIDDEN), whole),
            pl.BlockSpec((HIDDEN, 1), whole),
            pl.BlockSpec((1, 1), whole),
        ],
        out_specs=pl.BlockSpec((BLK, 1), row_blk),
        out_shape=jax.ShapeDtypeStruct((B, 1), jnp.float32),
    )(
        dm, t, d, nf,
        W1[0:TYPE_DIM],
        W1[TYPE_DIM:TYPE_DIM + DAY_DIM],
        W1[TYPE_DIM + DAY_DIM:TYPE_DIM + DAY_DIM + DOMAIN_DIM],
        w1nf,
        b1[None, :], W2, b2[None, :],
    )
    return out[:, 0]


# speedup vs baseline: 4.4212x; 1.2635x over previous
import functools

import jax
import jax.numpy as jnp
from jax import lax
from jax.experimental import pallas as pl
from jax.experimental.pallas import tpu as pltpu
from jax.experimental.pallas import tpu_sc as plsc

B = 16384
TYPE_VOCAB = 1000
DOMAIN_DIM = 16
TYPE_DIM = 8
DAY_DIM = 4
HIDDEN = 128

NC = 2
NS = 16
LANES = 16
NW = NC * NS
BPW = B // NW
GROUPS = BPW // LANES
DMA_CHUNK = 16


def _sc_gather_body(dom_t, typ_tab, day_tab, dom_id, typ_id, day_id,
                    dm_out, t_out, d_out,
                    didx_v, tiles_v, tid_v, did_v, ttab_v, dtab_v,
                    dmblk_v, tblk_v, dblk_v, sem):
    wid = lax.axis_index("s") * NC + lax.axis_index("c")
    base = wid * BPW

    pltpu.sync_copy(dom_id.at[pl.ds(base, BPW)], didx_v)
    pltpu.sync_copy(typ_id.at[pl.ds(base, BPW)], tid_v)
    pltpu.sync_copy(day_id.at[pl.ds(base, BPW)], did_v)
    pltpu.sync_copy(typ_tab, ttab_v)
    pltpu.sync_copy(day_tab, dtab_v)

    lane = lax.iota(jnp.int32, LANES)

    n_chunks = BPW // DMA_CHUNK

    def fire(ch):
        ids = didx_v[pl.ds(ch * DMA_CHUNK, DMA_CHUNK)]
        ks = lax.shift_right_logical(ids, 7)
        slot_base = (ch % 2) * DMA_CHUNK
        for j in range(DMA_CHUNK):
            off = pl.multiple_of(ks[j] * 128, 128)
            pltpu.async_copy(dom_t.at[:, pl.ds(off, 128)],
                             tiles_v.at[slot_base + j], sem)

    def extract(ch):
        ids = didx_v[pl.ds(ch * DMA_CHUNK, DMA_CHUNK)]
        ls = ids & 127
        slot_base = (ch % 2) * DMA_CHUNK
        for j in range(DMA_CHUNK):
            pltpu.make_async_copy(dom_t.at[:, pl.ds(0, 128)],
                                  tiles_v.at[slot_base + j], sem).wait()
        for j in range(DMA_CHUNK):
            i = ch * DMA_CHUNK + j
            v = plsc.load_gather(
                tiles_v, [jnp.broadcast_to(slot_base + j, (LANES,)),
                          lane, jnp.broadcast_to(ls[j], (LANES,))])
            dmblk_v[pl.ds(i * DOMAIN_DIM, DOMAIN_DIM)] = v

    def type_day_group(g):
        tids = tid_v[pl.ds(g * LANES, LANES)]
        tdst = (g * LANES + lane) * TYPE_DIM
        tsrc = tids * TYPE_DIM
        for j in range(TYPE_DIM):
            v = plsc.load_gather(ttab_v, [tsrc + j])
            plsc.store_scatter(tblk_v, [tdst + j], v)

        dids = did_v[pl.ds(g * LANES, LANES)]
        ddst = (g * LANES + lane) * DAY_DIM
        dsrc = dids * DAY_DIM
        for j in range(DAY_DIM):
            v = plsc.load_gather(dtab_v, [dsrc + j])
            plsc.store_scatter(dblk_v, [ddst + j], v)

    def pipe(ch, _):
        @pl.when(ch < n_chunks)
        def _():
            fire(ch)
            type_day_group(ch)

        @pl.when(ch > 0)
        def _():
            extract(ch - 1)

        return ()

    lax.fori_loop(0, n_chunks + 1, pipe, (), unroll=False)

    pltpu.sync_copy(dmblk_v,
                    dm_out.at[pl.ds(base * DOMAIN_DIM, BPW * DOMAIN_DIM)])
    pltpu.sync_copy(tblk_v, t_out.at[pl.ds(base * TYPE_DIM, BPW * TYPE_DIM)])
    pltpu.sync_copy(dblk_v, d_out.at[pl.ds(base * DAY_DIM, BPW * DAY_DIM)])


_sc_gather = functools.partial(
    pl.kernel,
    out_type=(
        jax.ShapeDtypeStruct((B * DOMAIN_DIM,), jnp.float32),
        jax.ShapeDtypeStruct((B * TYPE_DIM,), jnp.float32),
        jax.ShapeDtypeStruct((B * DAY_DIM,), jnp.float32),
    ),
    mesh=plsc.VectorSubcoreMesh(core_axis_name="c", subcore_axis_name="s"),
    compiler_params=pltpu.CompilerParams(needs_layout_passes=False,
                                         use_tc_tiling_on_sc=True),
    scratch_types=(
        pltpu.VMEM((BPW,), jnp.int32),
        pltpu.VMEM((2 * DMA_CHUNK, DOMAIN_DIM, 128), jnp.float32),
        pltpu.VMEM((BPW,), jnp.int32),
        pltpu.VMEM((BPW,), jnp.int32),
        pltpu.VMEM((TYPE_VOCAB * TYPE_DIM,), jnp.float32),
        pltpu.VMEM((8 * DAY_DIM,), jnp.float32),
        pltpu.VMEM((BPW * DOMAIN_DIM,), jnp.float32),
        pltpu.VMEM((BPW * TYPE_DIM,), jnp.float32),
        pltpu.VMEM((BPW * DAY_DIM,), jnp.float32),
        pltpu.SemaphoreType.DMA,
    ),
)(_sc_gather_body)


BLK = 2048


def _mlp_body(dm_ref, t_ref, d_ref, nf_ref,
              w1t_ref, w1d_ref, w1dm_ref, w1nf_ref,
              b1_ref, w2_ref, b2_ref, out_ref):
    h = jnp.dot(t_ref[...], w1t_ref[...], preferred_element_type=jnp.float32)
    h += jnp.dot(d_ref[...], w1d_ref[...], preferred_element_type=jnp.float32)
    h += jnp.dot(dm_ref[...], w1dm_ref[...], preferred_element_type=jnp.float32)
    h += jnp.dot(nf_ref[...], w1nf_ref[...], preferred_element_type=jnp.float32)
    h = jnp.maximum(h + b1_ref[...], 0.0)
    out = jnp.dot(h, w2_ref[...], preferred_element_type=jnp.float32)
    out_ref[...] = out + b2_ref[...]


def kernel(type_id, day_of_week_id, domain_id, hour_of_day, karma, descendants,
           type_table, day_table, domain_table, W1, b1, W2, b2):
    type_id = type_id.astype(jnp.int32)
    day_of_week_id = day_of_week_id.astype(jnp.int32)
    domain_id = domain_id.astype(jnp.int32)
    day_pad = jnp.pad(day_table, ((0, 8 - day_table.shape[0]), (0, 0)))

    dm, t, d = _sc_gather(domain_table.T,
                          type_table.reshape(-1), day_pad.reshape(-1),
                          domain_id, type_id, day_of_week_id)
    dm = dm.reshape(B, DOMAIN_DIM)
    t = t.reshape(B, TYPE_DIM)
    d = d.reshape(B, DAY_DIM)

    nf = jnp.stack([hour_of_day, karma, descendants,
                    jnp.zeros((B,), jnp.float32)], axis=1)
    w1nf = jnp.concatenate(
        [W1[28:31], jnp.zeros((1, HIDDEN), jnp.float32)], axis=0)

    row_blk = lambda i: (i, 0)
    whole = lambda i: (0, 0)
    out = pl.pallas_call(
        _mlp_body,
        grid=(B // BLK,),
        in_specs=[
            pl.BlockSpec((BLK, DOMAIN_DIM), row_blk),
            pl.BlockSpec((BLK, TYPE_DIM), row_blk),
            pl.BlockSpec((BLK, DAY_DIM), row_blk),
            pl.BlockSpec((BLK, 4), row_blk),
            pl.BlockSpec((TYPE_DIM, HIDDEN), whole),
            pl.BlockSpec((DAY_DIM, HIDDEN), whole),
            pl.BlockSpec((DOMAIN_DIM, HIDDEN), whole),
            pl.BlockSpec((4, HIDDEN), whole),
            pl.BlockSpec((1, HIDDEN), whole),
            pl.BlockSpec((HIDDEN, 1), whole),
            pl.BlockSpec((1, 1), whole),
        ],
        out_specs=pl.BlockSpec((BLK, 1), row_blk),
        out_shape=jax.ShapeDtypeStruct((B, 1), jnp.float32),
    )(
        dm, t, d, nf,
        W1[0:TYPE_DIM],
        W1[TYPE_DIM:TYPE_DIM + DAY_DIM],
        W1[TYPE_DIM + DAY_DIM:TYPE_DIM + DAY_DIM + DOMAIN_DIM],
        w1nf,
        b1[None, :], W2, b2[None, :],
    )
    return out[:, 0]
